# baseline (speedup 1.0000x reference)
import functools

import jax
import jax.numpy as jnp
from jax import lax
from jax.experimental import pallas as pl
from jax.experimental.pallas import tpu as pltpu
from jax.experimental.pallas import tpu_sc as plsc

_V, _B, _D, _E = 2, 32, 2048, 64
_H = 4 * _D
_TOPK = 8
_HBLK = 512
_NBLK = _H // _HBLK
_T = _V * _B
_LANES = 16
_NWORK = _T // _LANES


def _logits_kernel(x_ref, keys_ref, rw_ref, rbt_ref, lt_ref):
    x = x_ref[...]
    k = keys_ref[...]
    kx = lax.dot_general(k, x, (((1,), (1,)), ((), ())),
                         preferred_element_type=jnp.float32)
    rx = lax.dot_general(rw_ref[...], x, (((1,), (1,)), ((), ())),
                         preferred_element_type=jnp.float32)
    router = jnp.concatenate([rx[:_E, :_B], rx[_E:, _B:]], axis=1)
    xn = jnp.sum(x * x, axis=1)[None, :]
    kn = jnp.sum(k * k, axis=1)[:, None]
    lt_ref[...] = 2.0 * kx - xn - kn + router + rbt_ref[...]


def _mlp_kernel(x_ref, w1_ref, b1_ref, w2_ref, out_ref):
    i = pl.program_id(0)
    h = lax.dot_general(x_ref[...], w1_ref[...], (((1,), (1,)), ((), ())),
                        preferred_element_type=jnp.float32)
    h = h + b1_ref[...]
    h = 0.5 * h * (1.0 + lax.erf(h * jnp.float32(0.7071067811865476)))
    contrib = lax.dot_general(h, w2_ref[...], (((1,), (1,)), ((), ())),
                              preferred_element_type=jnp.float32)

    @pl.when(i == 0)
    def _init():
        out_ref[...] = contrib

    @pl.when(i > 0)
    def _accum():
        out_ref[...] += contrib


def _combine_kernel(acc_ref, b2_ref, w_ref, out_ref):
    y = (acc_ref[...] + b2_ref[...]) * w_ref[...]
    out_ref[...] = y[:_B, :] + y[_B:, :]


_sc_mesh = plsc.VectorSubcoreMesh(core_axis_name="c", subcore_axis_name="s")


@functools.partial(
    pl.kernel, mesh=_sc_mesh,
    out_type=jax.ShapeDtypeStruct((_T,), jnp.float32),
    scratch_types=[
        pltpu.VMEM((_E, _T), jnp.float32),
        pltpu.VMEM((_LANES,), jnp.float32),
    ],
)
def _topk_wsum_sc(lt_hbm, out_hbm, lt_v, w_v):
    wid = lax.axis_index("s") * 2 + lax.axis_index("c")

    @pl.when(wid < _NWORK)
    def _():
        base = wid * _LANES
        pltpu.sync_copy(lt_hbm, lt_v)
        ms = [jnp.full((_LANES,), -1e30, jnp.float32) for _ in range(_TOPK)]
        for e in range(_E):
            t = lt_v[e, pl.ds(base, _LANES)]
            for j in range(_TOPK):
                hi = jnp.maximum(ms[j], t)
                t = jnp.minimum(ms[j], t)
                ms[j] = hi
        s = jnp.zeros((_LANES,), jnp.float32)
        for j in range(_TOPK):
            s = s + jnp.exp(ms[j] - ms[0])
        w_v[...] = s / s
        pltpu.sync_copy(w_v, out_hbm.at[pl.ds(base, _LANES)])


def kernel(views, expert_keys, w1, b1, w2, b2, router_w, router_b):
    x = views.reshape(_T, _D)
    keys = expert_keys.reshape(_E, _D)
    rw = router_w.reshape(_V * _E, _D)
    rbt = jnp.concatenate(
        [jnp.broadcast_to(router_b[0][:, None], (_E, _B)),
         jnp.broadcast_to(router_b[1][:, None], (_E, _B))], axis=1)
    b1r = b1.reshape(1, _H)
    b2r = b2.reshape(1, _D)

    lt = pl.pallas_call(
        _logits_kernel,
        in_specs=[pl.BlockSpec((_T, _D), lambda: (0, 0)),
                  pl.BlockSpec((_E, _D), lambda: (0, 0)),
                  pl.BlockSpec((_V * _E, _D), lambda: (0, 0)),
                  pl.BlockSpec((_E, _T), lambda: (0, 0))],
        out_specs=pl.BlockSpec((_E, _T), lambda: (0, 0)),
        out_shape=jax.ShapeDtypeStruct((_E, _T), jnp.float32),
    )(x, keys, rw, rbt)

    wsum = _topk_wsum_sc(lt)

    acc = pl.pallas_call(
        _mlp_kernel,
        grid=(_NBLK,),
        in_specs=[
            pl.BlockSpec((_T, _D), lambda i: (0, 0)),
            pl.BlockSpec((_HBLK, _D), lambda i: (i, 0)),
            pl.BlockSpec((1, _HBLK), lambda i: (0, i)),
            pl.BlockSpec((_D, _HBLK), lambda i: (0, i)),
        ],
        out_specs=pl.BlockSpec((_T, _D), lambda i: (0, 0)),
        out_shape=jax.ShapeDtypeStruct((_T, _D), jnp.float32),
        compiler_params=pltpu.CompilerParams(
            dimension_semantics=("arbitrary",)),
    )(x, w1, b1r, w2)

    out = pl.pallas_call(
        _combine_kernel,
        in_specs=[pl.BlockSpec((_T, _D), lambda: (0, 0)),
                  pl.BlockSpec((1, _D), lambda: (0, 0)),
                  pl.BlockSpec((_T, 1), lambda: (0, 0))],
        out_specs=pl.BlockSpec((_B, _D), lambda: (0, 0)),
        out_shape=jax.ShapeDtypeStruct((_B, _D), jnp.float32),
    )(acc, b2r, wsum.reshape(_T, 1))
    return out.reshape(_B, 1, _D)


# speedup vs baseline: 1.3946x; 1.3946x over previous
import jax
import jax.numpy as jnp
from jax.experimental import pallas as pl
from jax.experimental.pallas import tpu as pltpu

_V, _B, _D, _E = 2, 32, 2048, 64
_H = 4 * _D
_TOPK = 8
_HBLK = 512
_NBLK = _H // _HBLK
_T = _V * _B


def _fused_mlp_kernel(x_ref, w1_ref, b1_ref, w2_ref, b2_ref,
                      keys_ref, rw_ref, rb_ref, out_ref, acc_ref, wsum_ref):
    i = pl.program_id(0)
    x = x_ref[...]
    h = jax.lax.dot_general(x, w1_ref[...], (((1,), (1,)), ((), ())),
                            preferred_element_type=jnp.float32)
    h = h + b1_ref[...]
    h = 0.5 * h * (1.0 + jax.lax.erf(h * jnp.float32(0.7071067811865476)))
    contrib = jax.lax.dot_general(h, w2_ref[...], (((1,), (1,)), ((), ())),
                                  preferred_element_type=jnp.float32)

    @pl.when(i == 0)
    def _init():
        acc_ref[...] = contrib
        k = keys_ref[...]
        xk = jax.lax.dot_general(x, k, (((1,), (1,)), ((), ())),
                                 preferred_element_type=jnp.float32)
        xr = jax.lax.dot_general(x, rw_ref[...], (((1,), (1,)), ((), ())),
                                 preferred_element_type=jnp.float32)
        router = jnp.concatenate([xr[:_B, :_E], xr[_B:, _E:]], axis=0)
        xn = jnp.sum(x * x, axis=1, keepdims=True)
        kn = jnp.sum(k * k, axis=1)[None, :]
        logits = 2.0 * xk - xn - kn + router + rb_ref[...]
        cur = logits
        m = jnp.max(cur, axis=1, keepdims=True)
        s = jnp.zeros((_T, 1), jnp.float32)
        for _ in range(_TOPK):
            mk = jnp.max(cur, axis=1, keepdims=True)
            s = s + jnp.exp(mk - m)
            cur = jnp.where(cur >= mk, jnp.float32(-1e30), cur)
        wsum_ref[...] = s / s

    @pl.when(i > 0)
    def _accum():
        acc_ref[...] += contrib

    @pl.when(i == _NBLK - 1)
    def _finish():
        y = (acc_ref[...] + b2_ref[...]) * wsum_ref[...]
        out_ref[...] = y[:_B, :] + y[_B:, :]


def kernel(views, expert_keys, w1, b1, w2, b2, router_w, router_b):
    x = views.reshape(_T, _D)
    keys = expert_keys.reshape(_E, _D)
    rw = router_w.reshape(_V * _E, _D)
    rb = jnp.concatenate([jnp.broadcast_to(router_b[0], (_B, _E)),
                          jnp.broadcast_to(router_b[1], (_B, _E))], axis=0)
    b1r = b1.reshape(1, _H)
    b2r = b2.reshape(1, _D)

    out = pl.pallas_call(
        _fused_mlp_kernel,
        grid=(_NBLK,),
        in_specs=[
            pl.BlockSpec((_T, _D), lambda i: (0, 0)),
            pl.BlockSpec((_HBLK, _D), lambda i: (i, 0)),
            pl.BlockSpec((1, _HBLK), lambda i: (0, i)),
            pl.BlockSpec((_D, _HBLK), lambda i: (0, i)),
            pl.BlockSpec((1, _D), lambda i: (0, 0)),
            pl.BlockSpec((_E, _D), lambda i: (0, 0)),
            pl.BlockSpec((_V * _E, _D), lambda i: (0, 0)),
            pl.BlockSpec((_T, _E), lambda i: (0, 0)),
        ],
        out_specs=pl.BlockSpec((_B, _D), lambda i: (0, 0)),
        out_shape=jax.ShapeDtypeStruct((_B, _D), jnp.float32),
        scratch_shapes=[pltpu.VMEM((_T, _D), jnp.float32),
                        pltpu.VMEM((_T, 1), jnp.float32)],
        compiler_params=pltpu.CompilerParams(
            dimension_semantics=("arbitrary",)),
    )(x, w1, b1r, w2, b2r, keys, rw, rb)
    return out.reshape(_B, 1, _D)
